# baseline (speedup 1.0000x reference)
import functools

import jax
import jax.numpy as jnp
from jax import lax
from jax.experimental import pallas as pl
from jax.experimental.pallas import tpu as pltpu
from jax.experimental.pallas import tpu_sc as plsc

N_EMB = 8192
EMB_DIM = 256
M_BLK = 256
WIN = (0, 2736, 5472, 8192)


def _vq_body(z_ref, zsq_ref, w_ref, wsq_ref, pen_ref, idx_ref, counts_ref,
             ppl_ref, *, num_blocks, n_total):
    z = z_ref[...]
    w = w_ref[...]
    mm = jax.lax.dot_general(z.astype(jnp.bfloat16), w.astype(jnp.bfloat16),
                             (((1,), (1,)), ((), ())),
                             preferred_element_type=jnp.float32)
    dist = (zsq_ref[...] + wsq_ref[...]) - 2.0 * mm
    colf = jax.lax.broadcasted_iota(jnp.int32, dist.shape, 1).astype(jnp.float32)

    def win_min(t):
        dw = dist + pen_ref[t, :][None, :]
        mn = jnp.min(dw, axis=1, keepdims=True)
        ix = jnp.min(jnp.where(dw == mn, colf, jnp.float32(N_EMB)), axis=1)
        return mn, ix

    v, idx = win_min(0)
    for t in (1, 2):
        mn, ix = win_min(t)
        vq = v.astype(jnp.bfloat16).astype(jnp.float32)
        repl = mn < vq
        idx = jnp.where(repl[:, 0], ix, idx)
        v = jnp.where(repl, mn, vq)

    idx_ref[...] = idx.astype(jnp.int32)[:, None]
    onehot = (colf == idx[:, None]).astype(jnp.float32)
    csum = jnp.sum(onehot, axis=0, keepdims=True)

    i = pl.program_id(0)

    @pl.when(i == 0)
    def _init():
        counts_ref[...] = csum

    @pl.when(i != 0)
    def _acc():
        counts_ref[...] = counts_ref[...] + csum

    @pl.when(i == num_blocks - 1)
    def _finish():
        p = counts_ref[...] * (1.0 / n_total)
        ent = jnp.sum(p * jnp.log(p + 1e-10), axis=(0, 1), keepdims=True)
        ppl_ref[...] = jnp.exp(-ent)


def _argmin_counts_ppl(flat, weight, zsq, wsq, pen):
    n, D = flat.shape
    K = weight.shape[0]
    num_blocks = n // M_BLK
    body = functools.partial(_vq_body, num_blocks=num_blocks, n_total=n)
    return pl.pallas_call(
        body,
        grid=(num_blocks,),
        in_specs=[
            pl.BlockSpec((M_BLK, D), lambda i: (i, 0)),
            pl.BlockSpec((M_BLK, 1), lambda i: (i, 0)),
            pl.BlockSpec((K, D), lambda i: (0, 0)),
            pl.BlockSpec((1, K), lambda i: (0, 0)),
            pl.BlockSpec((3, K), lambda i: (0, 0)),
        ],
        out_specs=[
            pl.BlockSpec((M_BLK, 1), lambda i: (i, 0)),
            pl.BlockSpec((1, K), lambda i: (0, 0)),
            pl.BlockSpec((1, 1), lambda i: (0, 0)),
        ],
        out_shape=[
            jax.ShapeDtypeStruct((n, 1), jnp.int32),
            jax.ShapeDtypeStruct((1, K), jnp.float32),
            jax.ShapeDtypeStruct((1, 1), jnp.float32),
        ],
    )(flat, zsq, weight, wsq, pen)


def _make_sc_gather(n, D):
    info = plsc.get_sparse_core_info()
    nw = info.num_cores * info.num_subcores
    b_per_w = n // nw
    chunk = 128
    mesh = plsc.VectorSubcoreMesh(core_axis_name="c", subcore_axis_name="s")

    @functools.partial(
        pl.kernel, mesh=mesh,
        out_type=jax.ShapeDtypeStruct((n, D), jnp.float32),
        scratch_types=[
            pltpu.VMEM((chunk,), jnp.int32),
            pltpu.VMEM((chunk, D), jnp.float32),
            pltpu.SemaphoreType.DMA,
        ],
    )
    def gather_k(table_hbm, idx_hbm, out_hbm, idx_v, rows_v, sem):
        wid = lax.axis_index("s") * info.num_cores + lax.axis_index("c")
        base = wid * b_per_w
        for c in range(b_per_w // chunk):
            off = base + c * chunk
            pltpu.sync_copy(idx_hbm.at[pl.ds(off, chunk)], idx_v)
            pltpu.async_copy(table_hbm.at[idx_v], rows_v, sem).wait()
            pltpu.sync_copy(rows_v, out_hbm.at[pl.ds(off, chunk)])

    return gather_k


@jax.jit
def kernel(f_emb, weight):
    K, D = weight.shape
    flat = f_emb.reshape(-1, D)
    n = flat.shape[0]
    zsq = jnp.sum(flat ** 2, axis=1, keepdims=True)
    wsq = jnp.sum(weight ** 2, axis=1).reshape(1, K)
    k_iota = jnp.arange(K)
    pen = jnp.stack([
        jnp.where((k_iota >= WIN[t]) & (k_iota < WIN[t + 1]), 0.0, jnp.inf)
        for t in range(3)
    ])

    idx, _counts, ppl = _argmin_counts_ppl(flat, weight, zsq, wsq, pen)

    wq = weight.astype(jnp.bfloat16).astype(jnp.float32)
    q = _make_sc_gather(n, D)(wq, idx.reshape(n))

    return (q.reshape(f_emb.shape), ppl.reshape(()), idx)


# speedup vs baseline: 1.1631x; 1.1631x over previous
import functools

import jax
import jax.numpy as jnp
from jax import lax
from jax.experimental import pallas as pl
from jax.experimental.pallas import tpu as pltpu
from jax.experimental.pallas import tpu_sc as plsc

N_EMB = 8192
EMB_DIM = 256
M_BLK = 256
WIN = (0, 2736, 5472, 8192)


def _vq_body(z_ref, zsq_ref, w_ref, wsq_ref, idx_ref, counts_ref,
             ppl_ref, *, num_blocks, n_total):
    z = z_ref[...]
    w = w_ref[...]
    mm = jax.lax.dot_general(z.astype(jnp.bfloat16), w.astype(jnp.bfloat16),
                             (((1,), (1,)), ((), ())),
                             preferred_element_type=jnp.float32)
    dist = (zsq_ref[...] + wsq_ref[...]) - 2.0 * mm

    def win_min(t):
        dw = jax.lax.slice_in_dim(dist, WIN[t], WIN[t + 1], axis=1)
        mn = jnp.min(dw, axis=1, keepdims=True)
        ix = jnp.argmin(dw, axis=1) + WIN[t]
        return mn, ix

    v, idx = win_min(0)
    for t in (1, 2):
        mn, ix = win_min(t)
        vq = v.astype(jnp.bfloat16).astype(jnp.float32)
        repl = mn < vq
        idx = jnp.where(repl[:, 0], ix, idx)
        v = jnp.where(repl, mn, vq)

    idx_ref[...] = idx[:, None]
    col = jax.lax.broadcasted_iota(jnp.int32, dist.shape, 1)
    onehot = (col == idx[:, None]).astype(jnp.float32)
    csum = jnp.sum(onehot, axis=0, keepdims=True)

    i = pl.program_id(0)

    @pl.when(i == 0)
    def _init():
        counts_ref[...] = csum

    @pl.when(i != 0)
    def _acc():
        counts_ref[...] = counts_ref[...] + csum

    @pl.when(i == num_blocks - 1)
    def _finish():
        p = counts_ref[...] * (1.0 / n_total)
        ent = jnp.sum(p * jnp.log(p + 1e-10), axis=(0, 1), keepdims=True)
        ppl_ref[...] = jnp.exp(-ent)


def _argmin_counts_ppl(flat, weight, zsq, wsq):
    n, D = flat.shape
    K = weight.shape[0]
    num_blocks = n // M_BLK
    body = functools.partial(_vq_body, num_blocks=num_blocks, n_total=n)
    return pl.pallas_call(
        body,
        grid=(num_blocks,),
        in_specs=[
            pl.BlockSpec((M_BLK, D), lambda i: (i, 0)),
            pl.BlockSpec((M_BLK, 1), lambda i: (i, 0)),
            pl.BlockSpec((K, D), lambda i: (0, 0)),
            pl.BlockSpec((1, K), lambda i: (0, 0)),
        ],
        out_specs=[
            pl.BlockSpec((M_BLK, 1), lambda i: (i, 0)),
            pl.BlockSpec((1, K), lambda i: (0, 0)),
            pl.BlockSpec((1, 1), lambda i: (0, 0)),
        ],
        out_shape=[
            jax.ShapeDtypeStruct((n, 1), jnp.int32),
            jax.ShapeDtypeStruct((1, K), jnp.float32),
            jax.ShapeDtypeStruct((1, 1), jnp.float32),
        ],
    )(flat, zsq, weight, wsq)


def _make_sc_gather(n, D):
    info = plsc.get_sparse_core_info()
    nw = info.num_cores * info.num_subcores
    b_per_w = n // nw
    chunk = 128
    mesh = plsc.VectorSubcoreMesh(core_axis_name="c", subcore_axis_name="s")

    @functools.partial(
        pl.kernel, mesh=mesh,
        out_type=jax.ShapeDtypeStruct((n, D), jnp.float32),
        scratch_types=[
            pltpu.VMEM((chunk,), jnp.int32),
            pltpu.VMEM((chunk, D), jnp.float32),
            pltpu.SemaphoreType.DMA,
        ],
    )
    def gather_k(table_hbm, idx_hbm, out_hbm, idx_v, rows_v, sem):
        wid = lax.axis_index("s") * info.num_cores + lax.axis_index("c")
        base = wid * b_per_w
        for c in range(b_per_w // chunk):
            off = base + c * chunk
            pltpu.sync_copy(idx_hbm.at[pl.ds(off, chunk)], idx_v)
            pltpu.async_copy(table_hbm.at[idx_v], rows_v, sem).wait()
            pltpu.sync_copy(rows_v, out_hbm.at[pl.ds(off, chunk)])

    return gather_k


@jax.jit
def kernel(f_emb, weight):
    K, D = weight.shape
    flat = f_emb.reshape(-1, D)
    n = flat.shape[0]
    zsq = jnp.sum(flat ** 2, axis=1, keepdims=True)
    wsq = jnp.sum(weight ** 2, axis=1).reshape(1, K)
    idx, _counts, ppl = _argmin_counts_ppl(flat, weight, zsq, wsq)

    wq = weight.astype(jnp.bfloat16).astype(jnp.float32)
    q = _make_sc_gather(n, D)(wq, idx.reshape(n))

    return (q.reshape(f_emb.shape), ppl.reshape(()), idx)


# speedup vs baseline: 1.4631x; 1.2579x over previous
import functools

import jax
import jax.numpy as jnp
from jax import lax
from jax.experimental import pallas as pl
from jax.experimental.pallas import tpu as pltpu
from jax.experimental.pallas import tpu_sc as plsc

N_EMB = 8192
EMB_DIM = 256
M_BLK = 256
WIN = (0, 2736, 5472, 8192)


def _vq_body(z_ref, zsq_ref, w_ref, wsq_ref, idx_ref, counts_ref,
             ppl_ref, *, num_blocks, n_total):
    z = z_ref[...]
    w = w_ref[...]
    mm = jax.lax.dot_general(z.astype(jnp.bfloat16), w.astype(jnp.bfloat16),
                             (((1,), (1,)), ((), ())),
                             preferred_element_type=jnp.float32)
    dist = (zsq_ref[...] + wsq_ref[...]) - 2.0 * mm
    col = jax.lax.broadcasted_iota(jnp.int32, dist.shape, 1)
    colf = col.astype(jnp.float32)

    def win_min(lo, hi):
        mask = (col >= lo) & (col < hi)
        dw = jnp.where(mask, dist, jnp.inf)
        mn = jnp.min(dw, axis=1, keepdims=True)
        ix = jnp.min(jnp.where(dw == mn, colf, jnp.float32(N_EMB)), axis=1)
        return mn, ix

    v, idx = win_min(WIN[0], WIN[1])
    for t in (1, 2):
        mn, ix = win_min(WIN[t], WIN[t + 1])
        vq = v.astype(jnp.bfloat16).astype(jnp.float32)
        repl = mn < vq
        idx = jnp.where(repl[:, 0], ix, idx)
        v = jnp.where(repl, mn, vq)

    idx_ref[...] = idx.astype(jnp.int32)[:, None]
    onehot = (colf == idx[:, None]).astype(jnp.float32)
    csum = jnp.sum(onehot, axis=0, keepdims=True)

    i = pl.program_id(0)

    @pl.when(i == 0)
    def _init():
        counts_ref[...] = csum

    @pl.when(i != 0)
    def _acc():
        counts_ref[...] = counts_ref[...] + csum

    @pl.when(i == num_blocks - 1)
    def _finish():
        p = counts_ref[...] * (1.0 / n_total)
        ent = jnp.sum(p * jnp.log(p + 1e-10), axis=(0, 1), keepdims=True)
        ppl_ref[...] = jnp.exp(-ent)


def _argmin_counts_ppl(flat, weight, zsq, wsq):
    n, D = flat.shape
    K = weight.shape[0]
    num_blocks = n // M_BLK
    body = functools.partial(_vq_body, num_blocks=num_blocks, n_total=n)
    return pl.pallas_call(
        body,
        grid=(num_blocks,),
        in_specs=[
            pl.BlockSpec((M_BLK, D), lambda i: (i, 0)),
            pl.BlockSpec((M_BLK, 1), lambda i: (i, 0)),
            pl.BlockSpec((K, D), lambda i: (0, 0)),
            pl.BlockSpec((1, K), lambda i: (0, 0)),
        ],
        out_specs=[
            pl.BlockSpec((M_BLK, 1), lambda i: (i, 0)),
            pl.BlockSpec((1, K), lambda i: (0, 0)),
            pl.BlockSpec((1, 1), lambda i: (0, 0)),
        ],
        out_shape=[
            jax.ShapeDtypeStruct((n, 1), jnp.int32),
            jax.ShapeDtypeStruct((1, K), jnp.float32),
            jax.ShapeDtypeStruct((1, 1), jnp.float32),
        ],
    )(flat, zsq, weight, wsq)


def _make_sc_gather(n, D):
    info = plsc.get_sparse_core_info()
    nw = info.num_cores * info.num_subcores
    b_per_w = n // nw
    chunk = 128
    mesh = plsc.VectorSubcoreMesh(core_axis_name="c", subcore_axis_name="s")

    @functools.partial(
        pl.kernel, mesh=mesh,
        out_type=jax.ShapeDtypeStruct((n, D), jnp.float32),
        scratch_types=[
            pltpu.VMEM((chunk,), jnp.int32),
            pltpu.VMEM((chunk, D), jnp.float32),
            pltpu.SemaphoreType.DMA,
        ],
    )
    def gather_k(table_hbm, idx_hbm, out_hbm, idx_v, rows_v, sem):
        wid = lax.axis_index("s") * info.num_cores + lax.axis_index("c")
        base = wid * b_per_w
        for c in range(b_per_w // chunk):
            off = base + c * chunk
            pltpu.sync_copy(idx_hbm.at[pl.ds(off, chunk)], idx_v)
            pltpu.async_copy(table_hbm.at[idx_v], rows_v, sem).wait()
            pltpu.sync_copy(rows_v, out_hbm.at[pl.ds(off, chunk)])

    return gather_k


@jax.jit
def kernel(f_emb, weight):
    K, D = weight.shape
    flat = f_emb.reshape(-1, D)
    n = flat.shape[0]
    zsq = jnp.sum(flat ** 2, axis=1, keepdims=True)
    wsq = jnp.sum(weight ** 2, axis=1).reshape(1, K)
    idx, _counts, ppl = _argmin_counts_ppl(flat, weight, zsq, wsq)

    wq = weight.astype(jnp.bfloat16).astype(jnp.float32)
    q = _make_sc_gather(n, D)(wq, idx.reshape(n))

    return (q.reshape(f_emb.shape), ppl.reshape(()), idx)


# speedup vs baseline: 1.5672x; 1.0712x over previous
import functools

import jax
import jax.numpy as jnp
from jax import lax
from jax.experimental import pallas as pl
from jax.experimental.pallas import tpu as pltpu
from jax.experimental.pallas import tpu_sc as plsc

N_EMB = 8192
EMB_DIM = 256
M_BLK = 256
WIN = (0, 2736, 5472, 8192)


def _vq_body(z_ref, zsq_ref, w_ref, wsq_ref, idx_ref, counts_ref,
             ppl_ref, *, num_blocks, n_total):
    z = z_ref[...]
    w = w_ref[...]
    mm = jax.lax.dot_general(z.astype(jnp.bfloat16), w.astype(jnp.bfloat16),
                             (((1,), (1,)), ((), ())),
                             preferred_element_type=jnp.float32)
    dist = (zsq_ref[...] + wsq_ref[...]) - 2.0 * mm
    col = jax.lax.broadcasted_iota(jnp.int32, dist.shape, 1)
    colf = col.astype(jnp.float32)

    def win_min(lo, hi):
        mask = (col >= lo) & (col < hi)
        dw = jnp.where(mask, dist, jnp.inf)
        mn = jnp.min(dw, axis=1, keepdims=True)
        ix = jnp.min(jnp.where(dw == mn, colf, jnp.float32(N_EMB)), axis=1)
        return mn, ix

    v, idx = win_min(WIN[0], WIN[1])
    for t in (1, 2):
        mn, ix = win_min(WIN[t], WIN[t + 1])
        vq = v.astype(jnp.bfloat16).astype(jnp.float32)
        repl = mn < vq
        idx = jnp.where(repl[:, 0], ix, idx)
        v = jnp.where(repl, mn, vq)

    idx_ref[...] = idx.astype(jnp.int32)[:, None]
    onehot = (colf == idx[:, None]).astype(jnp.bfloat16)
    ones_row = jnp.ones((1, onehot.shape[0]), jnp.bfloat16)
    csum = jax.lax.dot_general(ones_row, onehot, (((1,), (0,)), ((), ())),
                               preferred_element_type=jnp.float32)

    i = pl.program_id(0)

    @pl.when(i == 0)
    def _init():
        counts_ref[...] = csum

    @pl.when(i != 0)
    def _acc():
        counts_ref[...] = counts_ref[...] + csum

    @pl.when(i == num_blocks - 1)
    def _finish():
        p = counts_ref[...] * (1.0 / n_total)
        ent = jnp.sum(p * jnp.log(p + 1e-10), axis=(0, 1), keepdims=True)
        ppl_ref[...] = jnp.exp(-ent)


def _argmin_counts_ppl(flat, weight, zsq, wsq):
    n, D = flat.shape
    K = weight.shape[0]
    num_blocks = n // M_BLK
    body = functools.partial(_vq_body, num_blocks=num_blocks, n_total=n)
    return pl.pallas_call(
        body,
        grid=(num_blocks,),
        in_specs=[
            pl.BlockSpec((M_BLK, D), lambda i: (i, 0)),
            pl.BlockSpec((M_BLK, 1), lambda i: (i, 0)),
            pl.BlockSpec((K, D), lambda i: (0, 0)),
            pl.BlockSpec((1, K), lambda i: (0, 0)),
        ],
        out_specs=[
            pl.BlockSpec((M_BLK, 1), lambda i: (i, 0)),
            pl.BlockSpec((1, K), lambda i: (0, 0)),
            pl.BlockSpec((1, 1), lambda i: (0, 0)),
        ],
        out_shape=[
            jax.ShapeDtypeStruct((n, 1), jnp.int32),
            jax.ShapeDtypeStruct((1, K), jnp.float32),
            jax.ShapeDtypeStruct((1, 1), jnp.float32),
        ],
    )(flat, zsq, weight, wsq)


def _make_sc_gather(n, D):
    info = plsc.get_sparse_core_info()
    nw = info.num_cores * info.num_subcores
    b_per_w = n // nw
    chunk = 128
    mesh = plsc.VectorSubcoreMesh(core_axis_name="c", subcore_axis_name="s")

    @functools.partial(
        pl.kernel, mesh=mesh,
        out_type=jax.ShapeDtypeStruct((n, D), jnp.float32),
        scratch_types=[
            pltpu.VMEM((chunk,), jnp.int32),
            pltpu.VMEM((chunk, D), jnp.float32),
            pltpu.SemaphoreType.DMA,
        ],
    )
    def gather_k(table_hbm, idx_hbm, out_hbm, idx_v, rows_v, sem):
        wid = lax.axis_index("s") * info.num_cores + lax.axis_index("c")
        base = wid * b_per_w
        for c in range(b_per_w // chunk):
            off = base + c * chunk
            pltpu.sync_copy(idx_hbm.at[pl.ds(off, chunk)], idx_v)
            pltpu.async_copy(table_hbm.at[idx_v], rows_v, sem).wait()
            pltpu.sync_copy(rows_v, out_hbm.at[pl.ds(off, chunk)])

    return gather_k


@jax.jit
def kernel(f_emb, weight):
    K, D = weight.shape
    flat = f_emb.reshape(-1, D)
    n = flat.shape[0]
    zsq = jnp.sum(flat ** 2, axis=1, keepdims=True)
    wsq = jnp.sum(weight ** 2, axis=1).reshape(1, K)
    idx, _counts, ppl = _argmin_counts_ppl(flat, weight, zsq, wsq)

    wq = weight.astype(jnp.bfloat16).astype(jnp.float32)
    q = _make_sc_gather(n, D)(wq, idx.reshape(n))

    return (q.reshape(f_emb.shape), ppl.reshape(()), idx)


# speedup vs baseline: 1.5865x; 1.0123x over previous
import functools

import jax
import jax.numpy as jnp
from jax import lax
from jax.experimental import pallas as pl
from jax.experimental.pallas import tpu as pltpu
from jax.experimental.pallas import tpu_sc as plsc

N_EMB = 8192
EMB_DIM = 256
M_BLK = 512
WIN = (0, 2736, 5472, 8192)


def _vq_body(z_ref, zsq_ref, w_ref, wsq_ref, idx_ref, counts_ref,
             ppl_ref, *, num_blocks, n_total):
    z = z_ref[...]
    w = w_ref[...]
    mm = jax.lax.dot_general(z.astype(jnp.bfloat16), w.astype(jnp.bfloat16),
                             (((1,), (1,)), ((), ())),
                             preferred_element_type=jnp.float32)
    dist = (zsq_ref[...] + wsq_ref[...]) - 2.0 * mm
    col = jax.lax.broadcasted_iota(jnp.int32, dist.shape, 1)
    colf = col.astype(jnp.float32)

    def win_min(lo, hi):
        mask = (col >= lo) & (col < hi)
        dw = jnp.where(mask, dist, jnp.inf)
        mn = jnp.min(dw, axis=1, keepdims=True)
        ix = jnp.min(jnp.where(dw == mn, colf, jnp.float32(N_EMB)), axis=1)
        return mn, ix

    v, idx = win_min(WIN[0], WIN[1])
    for t in (1, 2):
        mn, ix = win_min(WIN[t], WIN[t + 1])
        vq = v.astype(jnp.bfloat16).astype(jnp.float32)
        repl = mn < vq
        idx = jnp.where(repl[:, 0], ix, idx)
        v = jnp.where(repl, mn, vq)

    idx_ref[...] = idx.astype(jnp.int32)[:, None]
    onehot = (colf == idx[:, None]).astype(jnp.bfloat16)
    ones_row = jnp.ones((1, onehot.shape[0]), jnp.bfloat16)
    csum = jax.lax.dot_general(ones_row, onehot, (((1,), (0,)), ((), ())),
                               preferred_element_type=jnp.float32)

    i = pl.program_id(0)

    @pl.when(i == 0)
    def _init():
        counts_ref[...] = csum

    @pl.when(i != 0)
    def _acc():
        counts_ref[...] = counts_ref[...] + csum

    @pl.when(i == num_blocks - 1)
    def _finish():
        p = counts_ref[...] * (1.0 / n_total)
        ent = jnp.sum(p * jnp.log(p + 1e-10), axis=(0, 1), keepdims=True)
        ppl_ref[...] = jnp.exp(-ent)


def _argmin_counts_ppl(flat, weight, zsq, wsq):
    n, D = flat.shape
    K = weight.shape[0]
    num_blocks = n // M_BLK
    body = functools.partial(_vq_body, num_blocks=num_blocks, n_total=n)
    return pl.pallas_call(
        body,
        grid=(num_blocks,),
        in_specs=[
            pl.BlockSpec((M_BLK, D), lambda i: (i, 0)),
            pl.BlockSpec((M_BLK, 1), lambda i: (i, 0)),
            pl.BlockSpec((K, D), lambda i: (0, 0)),
            pl.BlockSpec((1, K), lambda i: (0, 0)),
        ],
        out_specs=[
            pl.BlockSpec((M_BLK, 1), lambda i: (i, 0)),
            pl.BlockSpec((1, K), lambda i: (0, 0)),
            pl.BlockSpec((1, 1), lambda i: (0, 0)),
        ],
        out_shape=[
            jax.ShapeDtypeStruct((n, 1), jnp.int32),
            jax.ShapeDtypeStruct((1, K), jnp.float32),
            jax.ShapeDtypeStruct((1, 1), jnp.float32),
        ],
    )(flat, zsq, weight, wsq)


def _make_sc_gather(n, D):
    info = plsc.get_sparse_core_info()
    nw = info.num_cores * info.num_subcores
    b_per_w = n // nw
    chunk = 128
    mesh = plsc.VectorSubcoreMesh(core_axis_name="c", subcore_axis_name="s")

    @functools.partial(
        pl.kernel, mesh=mesh,
        out_type=jax.ShapeDtypeStruct((n, D), jnp.float32),
        scratch_types=[
            pltpu.VMEM((chunk,), jnp.int32),
            pltpu.VMEM((chunk, D), jnp.float32),
            pltpu.SemaphoreType.DMA,
        ],
    )
    def gather_k(table_hbm, idx_hbm, out_hbm, idx_v, rows_v, sem):
        wid = lax.axis_index("s") * info.num_cores + lax.axis_index("c")
        base = wid * b_per_w
        for c in range(b_per_w // chunk):
            off = base + c * chunk
            pltpu.sync_copy(idx_hbm.at[pl.ds(off, chunk)], idx_v)
            pltpu.async_copy(table_hbm.at[idx_v], rows_v, sem).wait()
            pltpu.sync_copy(rows_v, out_hbm.at[pl.ds(off, chunk)])

    return gather_k


@jax.jit
def kernel(f_emb, weight):
    K, D = weight.shape
    flat = f_emb.reshape(-1, D)
    n = flat.shape[0]
    zsq = jnp.sum(flat ** 2, axis=1, keepdims=True)
    wsq = jnp.sum(weight ** 2, axis=1).reshape(1, K)
    idx, _counts, ppl = _argmin_counts_ppl(flat, weight, zsq, wsq)

    wq = weight.astype(jnp.bfloat16).astype(jnp.float32)
    q = _make_sc_gather(n, D)(wq, idx.reshape(n))

    return (q.reshape(f_emb.shape), ppl.reshape(()), idx)
